# Initial kernel scaffold; baseline (speedup 1.0000x reference)
import jax
import jax.numpy as jnp
from jax.experimental import pallas as pl


def kernel(doc_ids, context_ids, sample_ids, paragraph_matrix, word_matrix, outputs_w):
    raise NotImplementedError("write your pallas kernel here")


# speedup vs baseline: 1.2772x; 1.2772x over previous
import jax
import jax.numpy as jnp
from jax import lax
from jax.experimental import pallas as pl
from jax.experimental.pallas import tpu as pltpu
from jax.experimental.pallas import tpu_sc as plsc

NC, NS = 2, 16
NW = NC * NS
TBLK = 512


def _shuf(v, perm):
    dnums = lax.GatherDimensionNumbers(
        offset_dims=(), collapsed_slice_dims=(0,), start_index_map=(0,))
    return lax.gather(v, perm[:, None], dnums, (1,),
                      mode=lax.GatherScatterMode.PROMISE_IN_BOUNDS)


def _hsum(v, perms):
    for p in perms:
        v = v + _shuf(v, p)
    return v


def _tr_body(w_ref, o_ref):
    o_ref[...] = w_ref[...].T


def _transpose(w):
    d, n = w.shape
    nblk = pl.cdiv(n, TBLK)
    return pl.pallas_call(
        _tr_body,
        grid=(nblk,),
        in_specs=[pl.BlockSpec((d, TBLK), lambda i: (0, i))],
        out_specs=pl.BlockSpec((TBLK, d), lambda i: (i, 0)),
        out_shape=jax.ShapeDtypeStruct((nblk * TBLK, d), jnp.float32),
    )(w)


def _make_sc_kernel(B, C, S, D):
    BPW = B // NW
    G1 = 4
    G2 = 8
    NCH = D // 16

    def body(doc_hbm, ctx_hbm, samp_hbm, par_hbm, word_hbm, wt_hbm, out_hbm,
             didx, cidx, sidx, par_v, inputs_v, wgrp, sgrp, out_v, sem):
        wid = lax.axis_index("s") * NC + lax.axis_index("c")
        base = pl.multiple_of(wid * BPW, BPW)

        pltpu.sync_copy(doc_hbm.at[pl.ds(base, BPW)], didx)
        pltpu.sync_copy(ctx_hbm.at[pl.ds(pl.multiple_of(wid * BPW * C, 8), BPW * C)], cidx)
        pltpu.sync_copy(samp_hbm.at[pl.ds(pl.multiple_of(wid * BPW * S, 8), BPW * S)], sidx)

        pltpu.async_copy(par_hbm.at[didx], par_v, sem).wait()

        iota16 = lax.iota(jnp.int32, 16)
        perms = [jnp.bitwise_xor(iota16, sh) for sh in (8, 4, 2, 1)]

        def p1(g, carry):
            st = pl.multiple_of(g * (G1 * C), 8)
            pltpu.async_copy(word_hbm.at[cidx.at[pl.ds(st, G1 * C)]], wgrp, sem).wait()
            for gl in range(G1):
                b = g * G1 + gl
                for ch in range(NCH):
                    sl = pl.ds(ch * 16, 16)
                    acc = par_v[b, sl]
                    for c in range(C):
                        acc = acc + wgrp[gl * C + c, sl]
                    inputs_v[b, sl] = acc
            return carry

        lax.fori_loop(0, BPW // G1, p1, 0)

        def p2(g, carry):
            st = pl.multiple_of(g * (G2 * S), 8)
            pltpu.async_copy(wt_hbm.at[sidx.at[pl.ds(st, G2 * S)]], sgrp, sem).wait()
            for gl in range(G2):
                b = g * G2 + gl
                ins = [inputs_v[b, pl.ds(ch * 16, 16)] for ch in range(NCH)]
                oacc = jnp.zeros((16,), jnp.float32)
                for s in range(S):
                    r = gl * S + s
                    acc = ins[0] * sgrp[r, pl.ds(0, 16)]
                    for ch in range(1, NCH):
                        acc = acc + ins[ch] * sgrp[r, pl.ds(ch * 16, 16)]
                    oacc = jnp.where(iota16 == s, _hsum(acc, perms), oacc)
                out_v[b] = oacc
            return carry

        lax.fori_loop(0, BPW // G2, p2, 0)

        pltpu.sync_copy(out_v, out_hbm.at[pl.ds(base, BPW)])

    mesh = plsc.VectorSubcoreMesh(
        core_axis_name="c", subcore_axis_name="s", num_cores=NC, num_subcores=NS
    )
    return pl.kernel(
        body,
        out_type=jax.ShapeDtypeStruct((B, S), jnp.float32),
        mesh=mesh,
        scratch_types=[
            pltpu.VMEM((BPW,), jnp.int32),
            pltpu.VMEM((BPW * C,), jnp.int32),
            pltpu.VMEM((BPW * S,), jnp.int32),
            pltpu.VMEM((BPW, D), jnp.float32),
            pltpu.VMEM((BPW, D), jnp.float32),
            pltpu.VMEM((G1 * C, D), jnp.float32),
            pltpu.VMEM((G2 * S, D), jnp.float32),
            pltpu.VMEM((BPW, S), jnp.float32),
            pltpu.SemaphoreType.DMA,
        ],
    )


def kernel(doc_ids, context_ids, sample_ids, paragraph_matrix, word_matrix, outputs_w):
    B = doc_ids.shape[0]
    C = context_ids.shape[1]
    S = sample_ids.shape[1]
    D = paragraph_matrix.shape[1]
    wt = _transpose(outputs_w)
    doc = doc_ids.astype(jnp.int32)
    ctx_flat = context_ids.astype(jnp.int32).reshape(-1)
    samp_flat = sample_ids.astype(jnp.int32).reshape(-1)
    sc = _make_sc_kernel(B, C, S, D)
    return sc(doc, ctx_flat, samp_flat, paragraph_matrix, word_matrix, wt)
